# baseline (device time: 40450 ns/iter reference)
import jax
import jax.numpy as jnp
from jax import lax
from jax.experimental import pallas as pl
from jax.experimental.pallas import tpu as pltpu

T = 256
D = 512
VP = 4096
V = 2 * VP
H = VP // 2
CS = 256
NX = H // CS


def kernel(x, W):
    def body(x_ref, w_ref, out_ref, xsend_sems, xrecv_sems,
             fsend_sems, frecv_sems):
        my_x = lax.axis_index("x")
        my_y = lax.axis_index("y")
        xnbr = (1 - my_x, my_y)
        ynbr = (my_x, 1 - my_y)

        bsem = pltpu.get_barrier_semaphore()
        for d in (xnbr, ynbr):
            pl.semaphore_signal(bsem, inc=1, device_id=d,
                                device_id_type=pl.DeviceIdType.MESH)
        pl.semaphore_wait(bsem, 2)

        xv = x_ref[...]
        my_lo = my_x * VP
        nbr_lo = (1 - my_x) * VP
        mine_half = my_y * H
        other_half = (1 - my_y) * H

        s_loc = jnp.zeros((T, 1), jnp.float32)
        xrdmas = []
        for c in range(NX):
            woff = mine_half + c * CS
            e = jnp.exp(jnp.dot(xv, w_ref[:, pl.ds(woff, CS)],
                                preferred_element_type=jnp.float32))
            s_loc = s_loc + jnp.sum(e, axis=1, keepdims=True)
            out_ref[:, pl.ds(my_lo + woff, CS)] = e
            rdma = pltpu.make_async_remote_copy(
                src_ref=out_ref.at[:, pl.ds(my_lo + woff, CS)],
                dst_ref=out_ref.at[:, pl.ds(my_lo + woff, CS)],
                send_sem=xsend_sems.at[c], recv_sem=xrecv_sems.at[c],
                device_id=xnbr, device_id_type=pl.DeviceIdType.MESH)
            rdma.start()
            xrdmas.append(rdma)
        s_nbr = jnp.zeros((T, 1), jnp.float32)
        frdmas = []
        for c in range(NX):
            xrdmas[c].wait_recv()
            lo = nbr_lo + mine_half + c * CS
            f = pltpu.make_async_remote_copy(
                src_ref=out_ref.at[:, pl.ds(lo, CS)],
                dst_ref=out_ref.at[:, pl.ds(lo, CS)],
                send_sem=fsend_sems.at[c], recv_sem=frecv_sems.at[c],
                device_id=ynbr, device_id_type=pl.DeviceIdType.MESH)
            f.start()
            frdmas.append(f)
            s_nbr = s_nbr + jnp.sum(out_ref[:, pl.ds(lo, CS)],
                                    axis=1, keepdims=True)
            woff = other_half + c * CS
            e = jnp.exp(jnp.dot(xv, w_ref[:, pl.ds(woff, CS)],
                                preferred_element_type=jnp.float32))
            s_loc = s_loc + jnp.sum(e, axis=1, keepdims=True)
            out_ref[:, pl.ds(my_lo + woff, CS)] = e
        for c in range(NX):
            lo = nbr_lo + other_half + c * CS
            recv = pltpu.make_async_remote_copy(
                src_ref=out_ref.at[:, pl.ds(lo, CS)],
                dst_ref=out_ref.at[:, pl.ds(lo, CS)],
                send_sem=fsend_sems.at[c], recv_sem=frecv_sems.at[c],
                device_id=ynbr, device_id_type=pl.DeviceIdType.MESH)
            recv.wait_recv()
            s_nbr = s_nbr + jnp.sum(out_ref[:, pl.ds(lo, CS)],
                                    axis=1, keepdims=True)
        for r in xrdmas:
            r.wait_send()
        for f in frdmas:
            f.wait_send()

        inv = 1.0 / (s_loc + s_nbr)
        out_ref[...] = out_ref[...] * inv

    return pl.pallas_call(
        body,
        out_shape=jax.ShapeDtypeStruct((T, V), jnp.float32),
        in_specs=[
            pl.BlockSpec(memory_space=pltpu.VMEM),
            pl.BlockSpec(memory_space=pltpu.VMEM),
        ],
        out_specs=pl.BlockSpec(memory_space=pltpu.VMEM),
        scratch_shapes=[
            pltpu.SemaphoreType.DMA((NX,)),
            pltpu.SemaphoreType.DMA((NX,)),
            pltpu.SemaphoreType.DMA((NX,)),
            pltpu.SemaphoreType.DMA((NX,)),
        ],
        compiler_params=pltpu.CompilerParams(collective_id=0),
    )(x, W)
